# Initial kernel scaffold; baseline (speedup 1.0000x reference)
import jax
import jax.numpy as jnp
from jax.experimental import pallas as pl


def kernel(target, y, now_iter):
    raise NotImplementedError("write your pallas kernel here")


# speedup vs baseline: 4.1485x; 4.1485x over previous
import jax
import jax.numpy as jnp
from jax.experimental import pallas as pl
from jax.experimental.pallas import tpu as pltpu

FINAL_ITER = 200000
THRESHOLD = 0.6

_ROWS = 128
_COLS = 100000
_BLOCK_ROWS = 8

_KS0 = 0
_KS1 = 12345
_KS2 = _KS0 ^ _KS1 ^ 0x1BD11BDA
_ROT0 = (13, 15, 26, 6)
_ROT1 = (17, 29, 16, 24)


def _select_body(t_ref, y_ref, thr_ref, o_ref):
    i = pl.program_id(0)
    rows, cols = t_ref.shape
    r = jax.lax.broadcasted_iota(jnp.uint32, (rows, cols), 0)
    c = jax.lax.broadcasted_iota(jnp.uint32, (rows, cols), 1)
    row0 = jnp.uint32(i * _BLOCK_ROWS)
    n = (row0 + r) * jnp.uint32(_COLS) + c

    ks = (jnp.uint32(_KS0), jnp.uint32(_KS1), jnp.uint32(_KS2))
    rotations = (_ROT0, _ROT1)

    x0 = ks[0]
    x1 = n + ks[1]
    for i_round in range(5):
        for d in rotations[i_round % 2]:
            x0 = x0 + x1
            x1 = (x1 << jnp.uint32(d)) | (x1 >> jnp.uint32(32 - d))
            x1 = x0 ^ x1
        x0 = x0 + ks[(i_round + 1) % 3]
        x1 = x1 + ks[(i_round + 2) % 3] + jnp.uint32(i_round + 1)

    bits = x0 ^ x1
    mask = (bits >> jnp.uint32(9)) < thr_ref[0]
    o_ref[...] = jnp.where(mask, y_ref[...], t_ref[...])


def kernel(target, y, now_iter):
    k = 1.0
    c = (k - THRESHOLD) / FINAL_ITER
    sampling_prob = jnp.maximum(THRESHOLD, k - c * now_iter)
    p = 1.0 - sampling_prob
    thr = jnp.ceil(p * 8388608.0).astype(jnp.uint32).reshape(1)

    grid = (_ROWS // _BLOCK_ROWS,)
    return pl.pallas_call(
        _select_body,
        grid=grid,
        in_specs=[
            pl.BlockSpec((_BLOCK_ROWS, _COLS), lambda i: (i, 0)),
            pl.BlockSpec((_BLOCK_ROWS, _COLS), lambda i: (i, 0)),
            pl.BlockSpec(memory_space=pltpu.SMEM),
        ],
        out_specs=pl.BlockSpec((_BLOCK_ROWS, _COLS), lambda i: (i, 0)),
        out_shape=jax.ShapeDtypeStruct((_ROWS, _COLS), jnp.float32),
    )(target, y, thr)


# speedup vs baseline: 4.2181x; 1.0168x over previous
import jax
import jax.numpy as jnp
from jax.experimental import pallas as pl
from jax.experimental.pallas import tpu as pltpu

FINAL_ITER = 200000
THRESHOLD = 0.6

_ROWS = 128
_COLS = 100000
_BLOCK_ROWS = 8

_KS0 = 0
_KS1 = 12345
_KS2 = _KS0 ^ _KS1 ^ 0x1BD11BDA
_ROT0 = (13, 15, 26, 6)
_ROT1 = (17, 29, 16, 24)


def _select_body(t_ref, y_ref, thr_ref, o_ref):
    i = pl.program_id(0)
    rows, cols = t_ref.shape
    r = jax.lax.broadcasted_iota(jnp.uint32, (rows, cols), 0)
    c = jax.lax.broadcasted_iota(jnp.uint32, (rows, cols), 1)
    row0 = jnp.uint32(i * _BLOCK_ROWS)
    n = (row0 + r) * jnp.uint32(_COLS) + c

    ks = (_KS0, _KS1, _KS2)
    rotations = (_ROT0, _ROT1)

    x0 = jnp.uint32(ks[0])
    x1 = n + jnp.uint32(ks[1])
    for i_round in range(5):
        for d in rotations[i_round % 2]:
            x0 = x0 + x1
            x1 = (x1 << jnp.uint32(d)) | (x1 >> jnp.uint32(32 - d))
            x1 = x0 ^ x1
        x0 = x0 + jnp.uint32(ks[(i_round + 1) % 3])
        x1 = x1 + jnp.uint32((ks[(i_round + 2) % 3] + i_round + 1) & 0xFFFFFFFF)

    bits = x0 ^ x1
    mask = bits < thr_ref[0]
    o_ref[...] = jnp.where(mask, y_ref[...], t_ref[...])


def kernel(target, y, now_iter):
    k = 1.0
    c = (k - THRESHOLD) / FINAL_ITER
    sampling_prob = jnp.maximum(THRESHOLD, k - c * now_iter)
    p = 1.0 - sampling_prob
    thr = (jnp.ceil(p * 8388608.0).astype(jnp.uint32) << 9).reshape(1)

    grid = (_ROWS // _BLOCK_ROWS,)
    return pl.pallas_call(
        _select_body,
        grid=grid,
        in_specs=[
            pl.BlockSpec((_BLOCK_ROWS, _COLS), lambda i: (i, 0)),
            pl.BlockSpec((_BLOCK_ROWS, _COLS), lambda i: (i, 0)),
            pl.BlockSpec(memory_space=pltpu.SMEM),
        ],
        out_specs=pl.BlockSpec((_BLOCK_ROWS, _COLS), lambda i: (i, 0)),
        out_shape=jax.ShapeDtypeStruct((_ROWS, _COLS), jnp.float32),
        compiler_params=pltpu.CompilerParams(
            dimension_semantics=("parallel",)),
    )(target, y, thr)
